# baseline (speedup 1.0000x reference)
import functools

import jax
import jax.numpy as jnp
from jax import lax
from jax.experimental import pallas as pl
from jax.experimental.pallas import tpu as pltpu
from jax.experimental.pallas import tpu_sc as plsc

_NUM_CORES = 2
_NUM_SUBCORES = 16
_NW = _NUM_CORES * _NUM_SUBCORES
_LANES = 16
_PBLOCK = 64


def _gather_sc_pairs(pair_flat, pidx, n_pairs, d2):
    pairs_per_w = n_pairs // _NW
    n_blocks = pairs_per_w // _PBLOCK
    groups_per_block = _PBLOCK // _LANES
    pvd = pair_flat.shape[0]
    mesh = plsc.VectorSubcoreMesh(
        core_axis_name="c",
        subcore_axis_name="s",
        num_cores=_NUM_CORES,
        num_subcores=_NUM_SUBCORES,
    )

    @functools.partial(
        pl.kernel,
        out_type=jax.ShapeDtypeStruct((n_pairs * d2,), jnp.float32),
        mesh=mesh,
        compiler_params=pltpu.CompilerParams(needs_layout_passes=False),
        scratch_types=[
            pltpu.VMEM((pvd,), jnp.float32),
            pltpu.VMEM((pairs_per_w,), jnp.int32),
            pltpu.VMEM((_PBLOCK * d2,), jnp.float32),
            pltpu.VMEM((_PBLOCK * d2,), jnp.float32),
            pltpu.SemaphoreType.DMA,
            pltpu.SemaphoreType.DMA,
        ],
    )
    def k(pair_hbm, pidx_hbm, out_hbm, pair_v, pidx_v, rows0, rows1, w0, w1):
        wid = lax.axis_index("s") * _NUM_CORES + lax.axis_index("c")
        base_w = wid * pairs_per_w
        wsem = (w0, w1)
        rowbuf = (rows0, rows1)

        pltpu.sync_copy(pair_hbm, pair_v)
        pltpu.sync_copy(pidx_hbm.at[pl.ds(base_w, pairs_per_w)], pidx_v)

        def w_desc(blk, b):
            return pltpu.make_async_copy(
                rowbuf[b],
                out_hbm.at[pl.ds((base_w + blk * _PBLOCK) * d2, _PBLOCK * d2)],
                wsem[b],
            )

        def compute(blk, b):
            ob = rowbuf[b]

            @plsc.parallel_loop(0, groups_per_block)
            def _group(g):
                srcs = pidx_v[pl.ds(blk * _PBLOCK + g * _LANES, _LANES)] * d2
                src = [srcs[j] for j in range(_LANES)]
                dst = [(g * _LANES + j) * d2 for j in range(_LANES)]
                for kk in range(d2 // _LANES):
                    for j in range(_LANES):
                        ob[pl.ds(dst[j] + kk * _LANES, _LANES)] = (
                            pair_v[pl.ds(src[j] + kk * _LANES, _LANES)]
                        )

        @pl.loop(0, n_blocks, step=2)
        def _body(i):
            for b in range(2):
                blk = i + b

                @pl.when(blk >= 2)
                def _():
                    w_desc(blk - 2, b).wait()

                compute(blk, b)
                w_desc(blk, b).start()

        w_desc(n_blocks - 2, 0).wait()
        w_desc(n_blocks - 1, 1).wait()

    return k(pair_flat, pidx)


def kernel(inputs, emb_table):
    b, s = inputs.shape
    v, d = emb_table.shape
    n = b * s
    pair_table = jnp.concatenate(
        [jnp.repeat(emb_table, v, axis=0), jnp.tile(emb_table, (v, 1))], axis=1
    )
    idx = inputs.reshape(-1)
    pidx = idx[0::2] * v + idx[1::2]
    out = _gather_sc_pairs(pair_table.reshape(-1), pidx, n // 2, 2 * d)
    return out.reshape(b, s, d)


# speedup vs baseline: 2.1495x; 2.1495x over previous
import functools

import jax
import jax.numpy as jnp
from jax import lax
from jax.experimental import pallas as pl
from jax.experimental.pallas import tpu as pltpu
from jax.experimental.pallas import tpu_sc as plsc

_NUM_CORES = 2
_NUM_SUBCORES = 16
_NW = _NUM_CORES * _NUM_SUBCORES
_LANES = 16
_BLOCK = 320


def _gather_sc(table_flat, idx_flat, n_rows, d):
    rows_per_w = n_rows // _NW
    n_blocks = rows_per_w // _BLOCK
    vd = table_flat.shape[0]
    mesh = plsc.VectorSubcoreMesh(
        core_axis_name="c",
        subcore_axis_name="s",
        num_cores=_NUM_CORES,
        num_subcores=_NUM_SUBCORES,
    )

    @functools.partial(
        pl.kernel,
        out_type=jax.ShapeDtypeStruct((n_rows * d,), jnp.float32),
        mesh=mesh,
        compiler_params=pltpu.CompilerParams(needs_layout_passes=False),
        scratch_types=[
            pltpu.VMEM((vd,), jnp.float32),
            pltpu.VMEM((rows_per_w,), jnp.int32),
            pltpu.VMEM((_BLOCK * d,), jnp.float32),
            pltpu.VMEM((_BLOCK * d,), jnp.float32),
            pltpu.SemaphoreType.DMA,
            pltpu.SemaphoreType.DMA,
        ],
    )
    def k(table_hbm, idx_hbm, out_hbm, table_v, idx_v, rows0, rows1, w0, w1):
        wid = lax.axis_index("s") * _NUM_CORES + lax.axis_index("c")
        base_w = wid * rows_per_w
        wsem = (w0, w1)
        rowbuf = (rows0, rows1)

        pltpu.sync_copy(table_hbm, table_v)
        pltpu.sync_copy(idx_hbm.at[pl.ds(base_w, rows_per_w)], idx_v)

        def w_desc(blk, b):
            return pltpu.make_async_copy(
                rowbuf[b],
                out_hbm.at[pl.ds((base_w + blk * _BLOCK) * d, _BLOCK * d)],
                wsem[b],
            )

        def compute(blk, b):
            ob = rowbuf[b]

            @plsc.parallel_loop(0, _BLOCK // _LANES, unroll=2)
            def _group(g):
                srcs = idx_v[pl.ds(blk * _BLOCK + g * _LANES, _LANES)] * d
                src = [srcs[j] for j in range(_LANES)]
                dst = [(g * _LANES + j) * d for j in range(_LANES)]
                for kk in range(d // _LANES):
                    for j in range(_LANES):
                        ob[pl.ds(dst[j] + kk * _LANES, _LANES)] = (
                            table_v[pl.ds(src[j] + kk * _LANES, _LANES)]
                        )

        @pl.loop(0, n_blocks, step=2)
        def _body(i):
            for b in range(2):
                blk = i + b

                @pl.when(blk >= 2)
                def _():
                    w_desc(blk - 2, b).wait()

                compute(blk, b)
                w_desc(blk, b).start()

        w_desc(n_blocks - 2, 0).wait()
        w_desc(n_blocks - 1, 1).wait()

    return k(table_flat, idx_flat)


def kernel(inputs, emb_table):
    b, s = inputs.shape
    _, d = emb_table.shape
    n = b * s
    out = _gather_sc(emb_table.reshape(-1), inputs.reshape(-1), n, d)
    return out.reshape(b, s, d)
